# baseline (speedup 1.0000x reference)
import functools

import jax
import jax.numpy as jnp
from jax import lax
from jax.experimental import pallas as pl
from jax.experimental.pallas import tpu as pltpu
from jax.experimental.pallas import tpu_sc as plsc

N = 10000
E = 320000
C = 128
R = 4
OUT = 128

NC = 2
NS = 16
NW = NC * NS

K = 128
CH_TOT = 2560
EPAD = CH_TOT * K
CHUNKS = CH_TOT // NW
NBUF = 2
GROUPS = CHUNKS // NBUF
ACCN = 10016
ROWS_PER_TILE = 624
ZERO_TAIL = ACCN - NS * ROWS_PER_TILE
OUT_TAIL = N - NS * ROWS_PER_TILE
NPAD = 10240
DEG_CHUNK = NPAD // NS


def _make_segsum(with_deg: bool):
    mesh = plsc.VectorSubcoreMesh(core_axis_name="c", subcore_axis_name="s")
    out_type = [jax.ShapeDtypeStruct((NC, N, C), jnp.float32)]
    scratch = []
    for _ in range(NBUF):
        scratch += [pltpu.VMEM((K,), jnp.int32),
                    pltpu.VMEM((K,), jnp.int32)]
    scratch += [pltpu.VMEM((K, C), jnp.float32) for _ in range(NBUF)]
    scratch += [pltpu.VMEM_SHARED((ACCN, C), jnp.float32)]
    scratch += [pltpu.SemaphoreType.DMA for _ in range(NBUF)]
    if with_deg:
        out_type.append(jax.ShapeDtypeStruct((NPAD,), jnp.float32))
        out_type.append(jax.ShapeDtypeStruct((NPAD,), jnp.float32))
        scratch += [
            pltpu.VMEM((K,), jnp.float32),
            pltpu.VMEM_SHARED((NPAD,), jnp.float32),
        ]

    @functools.partial(pl.kernel, out_type=out_type, mesh=mesh,
                       scratch_types=scratch)
    def segsum(h_hbm, src_hbm, dst_hbm, z2_hbm, z1_hbm, *rest):
        if with_deg:
            out_hbm, deg0_hbm, deg1_hbm, *rest2 = rest
        else:
            out_hbm, *rest2 = rest
        idx = [(rest2[2 * b], rest2[2 * b + 1]) for b in range(NBUF)]
        rows = rest2[2 * NBUF:3 * NBUF]
        acc = rest2[3 * NBUF]
        sems = rest2[3 * NBUF + 1:4 * NBUF + 1]
        if with_deg:
            ones_v, dacc = rest2[4 * NBUF + 1:]
        cid = lax.axis_index("c")
        sid = lax.axis_index("s")
        wid = sid * NC + cid

        r0 = sid * ROWS_PER_TILE
        pltpu.sync_copy(z2_hbm.at[pl.ds(r0, ROWS_PER_TILE)],
                        acc.at[pl.ds(r0, ROWS_PER_TILE)])

        @pl.when(sid == 0)
        def _():
            pltpu.sync_copy(z2_hbm.at[pl.ds(NS * ROWS_PER_TILE, ZERO_TAIL)],
                            acc.at[pl.ds(NS * ROWS_PER_TILE, ZERO_TAIL)])
        if with_deg:
            d0 = sid * DEG_CHUNK
            pltpu.sync_copy(z1_hbm.at[pl.ds(d0, DEG_CHUNK)],
                            dacc.at[pl.ds(d0, DEG_CHUNK)])
            for i in range(K // 16):
                ones_v[pl.ds(i * 16, 16)] = jnp.ones((16,), jnp.float32)
        plsc.subcore_barrier()

        base = wid * CHUNKS * K

        def load_idx(b, ci):
            pltpu.sync_copy(src_hbm.at[pl.ds(base + ci * K, K)], idx[b][0])
            pltpu.sync_copy(dst_hbm.at[pl.ds(base + ci * K, K)], idx[b][1])

        def body(ci, carry):
            load_idx(0, ci)
            pltpu.async_copy(h_hbm.at[idx[0][0]], rows[0], sems[0]).wait()
            pltpu.sync_copy(rows[0], acc.at[idx[0][1]], add=True)
            if with_deg:
                pltpu.sync_copy(ones_v, dacc.at[idx[0][1]], add=True)
            return carry

        lax.fori_loop(0, CHUNKS, body, 0)

        plsc.subcore_barrier()
        pltpu.sync_copy(acc.at[pl.ds(r0, ROWS_PER_TILE)],
                        out_hbm.at[cid, pl.ds(r0, ROWS_PER_TILE)])

        @pl.when(sid == 0)
        def _():
            pltpu.sync_copy(acc.at[pl.ds(NS * ROWS_PER_TILE, OUT_TAIL)],
                            out_hbm.at[cid, pl.ds(NS * ROWS_PER_TILE, OUT_TAIL)])
        if with_deg:
            @pl.when(cid == 0)
            def _():
                pltpu.sync_copy(dacc.at[pl.ds(d0, DEG_CHUNK)],
                                deg0_hbm.at[pl.ds(d0, DEG_CHUNK)])

            @pl.when(cid == 1)
            def _():
                pltpu.sync_copy(dacc.at[pl.ds(d0, DEG_CHUNK)],
                                deg1_hbm.at[pl.ds(d0, DEG_CHUNK)])

    return segsum


_SEGSUM_CACHE = {}


def _segsum_kernel(with_deg: bool):
    if with_deg not in _SEGSUM_CACHE:
        _SEGSUM_CACHE[with_deg] = _make_segsum(with_deg)
    return _SEGSUM_CACHE[with_deg]


BLK = 1000


def _encode_body(x_ref, aa_ref, ab_ref, da_ref, db_ref, wd_ref, wb_ref, b_ref,
                 out_ref):
    deg = da_ref[...] + db_ref[...]
    agg = aa_ref[...] + ab_ref[...]
    h = (deg * (jnp.dot(x_ref[...], wd_ref[...],
                        preferred_element_type=jnp.float32) + b_ref[...])
         + jnp.dot(agg, wb_ref[...], preferred_element_type=jnp.float32))
    out_ref[...] = jnp.where(h > 0, h, 0.2 * h)


def _stage2_body(h_ref, aa_ref, ab_ref, da_ref, db_ref, noise_ref,
                 wmd_ref, wmb_ref, bm_ref, wvd_ref, wvb_ref, bv_ref,
                 wdec_ref, bdec_ref, out_ref):
    deg = da_ref[...] + db_ref[...]
    agg = aa_ref[...] + ab_ref[...]
    h = h_ref[...]
    for r in range(R):
        m = deg * (jnp.dot(h, wmd_ref[r], preferred_element_type=jnp.float32)
                   + bm_ref[r]) + jnp.dot(agg, wmb_ref[r],
                                          preferred_element_type=jnp.float32)
        v = deg * (jnp.dot(h, wvd_ref[r], preferred_element_type=jnp.float32)
                   + bv_ref[r]) + jnp.dot(agg, wvb_ref[r],
                                          preferred_element_type=jnp.float32)
        z = m + noise_ref[:, r, :] * jnp.exp(0.5 * v)
        out_ref[:, r, :] = (jnp.dot(z, wdec_ref[...],
                                    preferred_element_type=jnp.float32)
                            + bdec_ref[...])


def _shuffled_weights(W, b):
    top, bot = W[:C], W[C:]
    wd = (top - bot).reshape(C, C, R).transpose(2, 0, 1)
    wb = bot.reshape(C, C, R).transpose(2, 0, 1)
    bs = b.reshape(C, R).T.reshape(R, 1, C)
    return wd, wb, bs


def kernel(x, edge_index, W_enc, b_enc, W_mean, b_mean, W_logvar, b_logvar,
           W_dec, b_dec):
    src_parts, dst_parts, start = [], [], 0
    for w in range(NW):
        n = 79 if w < 4 else 78
        src_parts += [edge_index[0, start * K:(start + n) * K],
                      jnp.zeros(((CHUNKS - n) * K,), edge_index.dtype)]
        dst_parts += [edge_index[1, start * K:(start + n) * K],
                      jnp.full(((CHUNKS - n) * K,), N + w // NC,
                               edge_index.dtype)]
        start += n
    src = jnp.concatenate(src_parts).astype(jnp.int32)
    dst = jnp.concatenate(dst_parts).astype(jnp.int32)
    z2 = jnp.zeros((ACCN, C), jnp.float32)
    z1 = jnp.zeros((NPAD,), jnp.float32)

    agg1, deg0, deg1 = _segsum_kernel(True)(x, src, dst, z2, z1)
    deg_a = deg0[:N].reshape(N, 1)
    deg_b = deg1[:N].reshape(N, 1)

    wd_enc = W_enc[:C] - W_enc[C:]
    wb_enc = W_enc[C:]
    full = lambda s: pl.BlockSpec(s, lambda i: (0,) * len(s))
    rows = lambda s: pl.BlockSpec(s, lambda i: (i,) + (0,) * (len(s) - 1))
    h1 = pl.pallas_call(
        _encode_body,
        grid=(N // BLK,),
        in_specs=[rows((BLK, C)), rows((BLK, C)), rows((BLK, C)),
                  rows((BLK, 1)), rows((BLK, 1)),
                  full((C, C)), full((C, C)), full((1, C))],
        out_specs=rows((BLK, C)),
        out_shape=jax.ShapeDtypeStruct((N, C), jnp.float32),
    )(x, agg1[0], agg1[1], deg_a, deg_b, wd_enc, wb_enc, b_enc.reshape(1, C))

    agg2 = _segsum_kernel(False)(h1, src, dst, z2, z1)
    if isinstance(agg2, (list, tuple)):
        agg2 = agg2[0]

    wmd, wmb, bm = _shuffled_weights(W_mean, b_mean)
    wvd, wvb, bv = _shuffled_weights(W_logvar, b_logvar)
    noise = jax.random.normal(jax.random.key(42), (N * R, OUT),
                              jnp.float32).reshape(N, R, OUT)

    out3 = pl.pallas_call(
        _stage2_body,
        grid=(N // BLK,),
        in_specs=[rows((BLK, C)), rows((BLK, C)), rows((BLK, C)),
                  rows((BLK, 1)), rows((BLK, 1)), rows((BLK, R, C)),
                  full((R, C, C)), full((R, C, C)), full((R, 1, C)),
                  full((R, C, C)), full((R, C, C)), full((R, 1, C)),
                  full((C, OUT)), full((1, OUT))],
        out_specs=rows((BLK, R, OUT)),
        out_shape=jax.ShapeDtypeStruct((N, R, OUT), jnp.float32),
    )(h1, agg2[0], agg2[1], deg_a, deg_b, noise,
      wmd, wmb, bm, wvd, wvb, bv, W_dec, b_dec.reshape(1, OUT))

    return out3.reshape(N * R, OUT)


# speedup vs baseline: 2.2017x; 2.2017x over previous
import functools

import jax
import jax.numpy as jnp
from jax import lax
from jax.experimental import pallas as pl
from jax.experimental.pallas import tpu as pltpu
from jax.experimental.pallas import tpu_sc as plsc

N = 10000
E = 320000
C = 128
R = 4
OUT = 128

NC = 2
NS = 16
NW = NC * NS

K = 128
CH_TOT = 2560
EPAD = CH_TOT * K
CHUNKS = CH_TOT // NW
NBUF = 2
GROUPS = CHUNKS // NBUF
ACCN = N + K
ROWS_PER_TILE = 624
ROWS_TAIL = N - NS * ROWS_PER_TILE
NPAD = 10240
DEG_CHUNK = NPAD // NS


def _make_segsum(with_deg: bool):
    mesh = plsc.VectorSubcoreMesh(core_axis_name="c", subcore_axis_name="s")
    out_type = [jax.ShapeDtypeStruct((NC, N, C), jnp.float32)]
    scratch = []
    for _ in range(NBUF):
        scratch += [pltpu.VMEM((K,), jnp.int32),
                    pltpu.VMEM((K,), jnp.int32)]
    scratch += [pltpu.VMEM((K, C), jnp.float32) for _ in range(NBUF)]
    scratch += [pltpu.VMEM_SHARED((ACCN, C), jnp.float32)]
    scratch += [pltpu.SemaphoreType.DMA for _ in range(NBUF)]
    if with_deg:
        out_type.append(jax.ShapeDtypeStruct((NPAD,), jnp.float32))
        out_type.append(jax.ShapeDtypeStruct((NPAD,), jnp.float32))
        scratch += [
            pltpu.VMEM((K,), jnp.float32),
            pltpu.VMEM_SHARED((NPAD,), jnp.float32),
        ]

    @functools.partial(pl.kernel, out_type=out_type, mesh=mesh,
                       scratch_types=scratch)
    def segsum(h_hbm, src_hbm, dst_hbm, z2_hbm, z1_hbm, *rest):
        if with_deg:
            out_hbm, deg0_hbm, deg1_hbm, *rest2 = rest
        else:
            out_hbm, *rest2 = rest
        idx = [(rest2[2 * b], rest2[2 * b + 1]) for b in range(NBUF)]
        rows = rest2[2 * NBUF:3 * NBUF]
        acc = rest2[3 * NBUF]
        sems = rest2[3 * NBUF + 1:4 * NBUF + 1]
        if with_deg:
            ones_v, dacc = rest2[4 * NBUF + 1:]
        cid = lax.axis_index("c")
        sid = lax.axis_index("s")
        wid = sid * NC + cid

        r0 = sid * ROWS_PER_TILE
        pltpu.sync_copy(z2_hbm.at[pl.ds(r0, ROWS_PER_TILE)],
                        acc.at[pl.ds(r0, ROWS_PER_TILE)])

        @pl.when(sid == 0)
        def _():
            pltpu.sync_copy(z2_hbm.at[pl.ds(NS * ROWS_PER_TILE, ROWS_TAIL)],
                            acc.at[pl.ds(NS * ROWS_PER_TILE, ROWS_TAIL)])
        if with_deg:
            d0 = sid * DEG_CHUNK
            pltpu.sync_copy(z1_hbm.at[pl.ds(d0, DEG_CHUNK)],
                            dacc.at[pl.ds(d0, DEG_CHUNK)])
            for i in range(K // 16):
                ones_v[pl.ds(i * 16, 16)] = jnp.ones((16,), jnp.float32)
        plsc.subcore_barrier()

        base = wid * CHUNKS * K

        def load_idx(b, ci):
            pltpu.sync_copy(src_hbm.at[pl.ds(base + ci * K, K)], idx[b][0])
            pltpu.sync_copy(dst_hbm.at[pl.ds(base + ci * K, K)], idx[b][1])

        for b in range(NBUF):
            load_idx(b, b)
            pltpu.async_copy(h_hbm.at[idx[b][0]], rows[b], sems[b])

        def group(g, carry):
            for b in range(NBUF):
                ci = g * NBUF + b
                pltpu.make_async_copy(h_hbm.at[pl.ds(0, K)], rows[b],
                                      sems[b]).wait()
                pltpu.sync_copy(rows[b], acc.at[idx[b][1]], add=True)
                if with_deg:
                    pltpu.sync_copy(ones_v, dacc.at[idx[b][1]], add=True)

                @pl.when(ci + NBUF < CHUNKS)
                def _(b=b, ci=ci):
                    load_idx(b, ci + NBUF)
                    pltpu.async_copy(h_hbm.at[idx[b][0]], rows[b], sems[b])
            return carry

        lax.fori_loop(0, GROUPS, group, 0)

        plsc.subcore_barrier()
        pltpu.sync_copy(acc.at[pl.ds(r0, ROWS_PER_TILE)],
                        out_hbm.at[cid, pl.ds(r0, ROWS_PER_TILE)])

        @pl.when(sid == 0)
        def _():
            pltpu.sync_copy(acc.at[pl.ds(NS * ROWS_PER_TILE, ROWS_TAIL)],
                            out_hbm.at[cid, pl.ds(NS * ROWS_PER_TILE, ROWS_TAIL)])
        if with_deg:
            @pl.when(cid == 0)
            def _():
                pltpu.sync_copy(dacc.at[pl.ds(d0, DEG_CHUNK)],
                                deg0_hbm.at[pl.ds(d0, DEG_CHUNK)])

            @pl.when(cid == 1)
            def _():
                pltpu.sync_copy(dacc.at[pl.ds(d0, DEG_CHUNK)],
                                deg1_hbm.at[pl.ds(d0, DEG_CHUNK)])

    return segsum


_SEGSUM_CACHE = {}


def _segsum_kernel(with_deg: bool):
    if with_deg not in _SEGSUM_CACHE:
        _SEGSUM_CACHE[with_deg] = _make_segsum(with_deg)
    return _SEGSUM_CACHE[with_deg]


BLK = 1000


def _encode_body(x_ref, aa_ref, ab_ref, da_ref, db_ref, wd_ref, wb_ref, b_ref,
                 out_ref):
    deg = da_ref[...] + db_ref[...]
    agg = aa_ref[...] + ab_ref[...]
    h = (deg * (jnp.dot(x_ref[...], wd_ref[...],
                        preferred_element_type=jnp.float32) + b_ref[...])
         + jnp.dot(agg, wb_ref[...], preferred_element_type=jnp.float32))
    out_ref[...] = jnp.where(h > 0, h, 0.2 * h)


def _stage2_body(h_ref, aa_ref, ab_ref, da_ref, db_ref, noise_ref,
                 wmd_ref, wmb_ref, bm_ref, wvd_ref, wvb_ref, bv_ref,
                 wdec_ref, bdec_ref, out_ref):
    deg = da_ref[...] + db_ref[...]
    agg = aa_ref[...] + ab_ref[...]
    h = h_ref[...]
    for r in range(R):
        m = deg * (jnp.dot(h, wmd_ref[r], preferred_element_type=jnp.float32)
                   + bm_ref[r]) + jnp.dot(agg, wmb_ref[r],
                                          preferred_element_type=jnp.float32)
        v = deg * (jnp.dot(h, wvd_ref[r], preferred_element_type=jnp.float32)
                   + bv_ref[r]) + jnp.dot(agg, wvb_ref[r],
                                          preferred_element_type=jnp.float32)
        z = m + noise_ref[:, r, :] * jnp.exp(0.5 * v)
        out_ref[:, r, :] = (jnp.dot(z, wdec_ref[...],
                                    preferred_element_type=jnp.float32)
                            + bdec_ref[...])


def _shuffled_weights(W, b):
    top, bot = W[:C], W[C:]
    wd = (top - bot).reshape(C, C, R).transpose(2, 0, 1)
    wb = bot.reshape(C, C, R).transpose(2, 0, 1)
    bs = b.reshape(C, R).T.reshape(R, 1, C)
    return wd, wb, bs


def kernel(x, edge_index, W_enc, b_enc, W_mean, b_mean, W_logvar, b_logvar,
           W_dec, b_dec):
    pad_src = jnp.arange(K, dtype=edge_index.dtype)
    pad_dst = jnp.arange(N, N + K, dtype=edge_index.dtype)
    src_parts, dst_parts, start = [], [], 0
    for w in range(NW):
        n = 79 if w < 4 else 78
        src_parts += [edge_index[0, start * K:(start + n) * K]]
        dst_parts += [edge_index[1, start * K:(start + n) * K]]
        src_parts += [pad_src] * (CHUNKS - n)
        dst_parts += [pad_dst] * (CHUNKS - n)
        start += n
    src = jnp.concatenate(src_parts).astype(jnp.int32)
    dst = jnp.concatenate(dst_parts).astype(jnp.int32)
    z2 = jnp.zeros((N, C), jnp.float32)
    z1 = jnp.zeros((NPAD,), jnp.float32)

    agg1, deg0, deg1 = _segsum_kernel(True)(x, src, dst, z2, z1)
    deg_a = deg0[:N].reshape(N, 1)
    deg_b = deg1[:N].reshape(N, 1)

    wd_enc = W_enc[:C] - W_enc[C:]
    wb_enc = W_enc[C:]
    full = lambda s: pl.BlockSpec(s, lambda i: (0,) * len(s))
    rows = lambda s: pl.BlockSpec(s, lambda i: (i,) + (0,) * (len(s) - 1))
    h1 = pl.pallas_call(
        _encode_body,
        grid=(N // BLK,),
        in_specs=[rows((BLK, C)), rows((BLK, C)), rows((BLK, C)),
                  rows((BLK, 1)), rows((BLK, 1)),
                  full((C, C)), full((C, C)), full((1, C))],
        out_specs=rows((BLK, C)),
        out_shape=jax.ShapeDtypeStruct((N, C), jnp.float32),
    )(x, agg1[0], agg1[1], deg_a, deg_b, wd_enc, wb_enc, b_enc.reshape(1, C))

    agg2 = _segsum_kernel(False)(h1, src, dst, z2, z1)
    if isinstance(agg2, (list, tuple)):
        agg2 = agg2[0]

    wmd, wmb, bm = _shuffled_weights(W_mean, b_mean)
    wvd, wvb, bv = _shuffled_weights(W_logvar, b_logvar)
    noise = jax.random.normal(jax.random.key(42), (N * R, OUT),
                              jnp.float32).reshape(N, R, OUT)

    out3 = pl.pallas_call(
        _stage2_body,
        grid=(N // BLK,),
        in_specs=[rows((BLK, C)), rows((BLK, C)), rows((BLK, C)),
                  rows((BLK, 1)), rows((BLK, 1)), rows((BLK, R, C)),
                  full((R, C, C)), full((R, C, C)), full((R, 1, C)),
                  full((R, C, C)), full((R, C, C)), full((R, 1, C)),
                  full((C, OUT)), full((1, OUT))],
        out_specs=rows((BLK, R, OUT)),
        out_shape=jax.ShapeDtypeStruct((N, R, OUT), jnp.float32),
    )(h1, agg2[0], agg2[1], deg_a, deg_b, noise,
      wmd, wmb, bm, wvd, wvb, bv, W_dec, b_dec.reshape(1, OUT))

    return out3.reshape(N * R, OUT)


# speedup vs baseline: 2.3286x; 1.0576x over previous
import functools

import jax
import jax.numpy as jnp
from jax import lax
from jax.experimental import pallas as pl
from jax.experimental.pallas import tpu as pltpu
from jax.experimental.pallas import tpu_sc as plsc

N = 10000
E = 320000
C = 128
R = 4
OUT = 128

NC = 2
NS = 16
NW = NC * NS

K = 128
CH_TOT = 2560
EPAD = CH_TOT * K
CHUNKS = CH_TOT // NW
NBUF = 2
GROUPS = CHUNKS // NBUF
ACCN = N + K
ROWS_PER_TILE = 624
ROWS_TAIL = N - NS * ROWS_PER_TILE
NPAD = 10240
DEG_CHUNK = NPAD // NS


def _make_segsum(with_deg: bool):
    mesh = plsc.VectorSubcoreMesh(core_axis_name="c", subcore_axis_name="s")
    out_type = [jax.ShapeDtypeStruct((NC, N, C), jnp.float32)]
    scratch = []
    for _ in range(NBUF):
        scratch += [pltpu.VMEM((K,), jnp.int32),
                    pltpu.VMEM((K,), jnp.int32)]
    scratch += [pltpu.VMEM((K, C), jnp.float32) for _ in range(NBUF)]
    scratch += [pltpu.VMEM_SHARED((ACCN, C), jnp.float32)]
    scratch += [pltpu.SemaphoreType.DMA for _ in range(2 * NBUF)]
    if with_deg:
        out_type.append(jax.ShapeDtypeStruct((NPAD,), jnp.float32))
        out_type.append(jax.ShapeDtypeStruct((NPAD,), jnp.float32))
        scratch += [
            pltpu.VMEM((K,), jnp.float32),
            pltpu.VMEM_SHARED((NPAD,), jnp.float32),
        ]

    @functools.partial(pl.kernel, out_type=out_type, mesh=mesh,
                       scratch_types=scratch)
    def segsum(h_hbm, src_hbm, dst_hbm, z2_hbm, z1_hbm, *rest):
        if with_deg:
            out_hbm, deg0_hbm, deg1_hbm, *rest2 = rest
        else:
            out_hbm, *rest2 = rest
        idx = [(rest2[2 * b], rest2[2 * b + 1]) for b in range(NBUF)]
        rows = rest2[2 * NBUF:3 * NBUF]
        acc = rest2[3 * NBUF]
        sems = rest2[3 * NBUF + 1:4 * NBUF + 1]
        isems = rest2[4 * NBUF + 1:5 * NBUF + 1]
        if with_deg:
            ones_v, dacc = rest2[5 * NBUF + 1:]
        cid = lax.axis_index("c")
        sid = lax.axis_index("s")
        wid = sid * NC + cid

        r0 = sid * ROWS_PER_TILE
        pltpu.sync_copy(z2_hbm.at[pl.ds(r0, ROWS_PER_TILE)],
                        acc.at[pl.ds(r0, ROWS_PER_TILE)])

        @pl.when(sid == 0)
        def _():
            pltpu.sync_copy(z2_hbm.at[pl.ds(NS * ROWS_PER_TILE, ROWS_TAIL)],
                            acc.at[pl.ds(NS * ROWS_PER_TILE, ROWS_TAIL)])
        if with_deg:
            d0 = sid * DEG_CHUNK
            pltpu.sync_copy(z1_hbm.at[pl.ds(d0, DEG_CHUNK)],
                            dacc.at[pl.ds(d0, DEG_CHUNK)])
            for i in range(K // 16):
                ones_v[pl.ds(i * 16, 16)] = jnp.ones((16,), jnp.float32)
        plsc.subcore_barrier()

        base = wid * CHUNKS * K

        def load_idx(b, ci):
            pltpu.sync_copy(src_hbm.at[pl.ds(base + ci * K, K)], idx[b][0])
            pltpu.async_copy(dst_hbm.at[pl.ds(base + ci * K, K)], idx[b][1],
                             isems[b])

        for b in range(NBUF):
            load_idx(b, b)
            pltpu.async_copy(h_hbm.at[idx[b][0]], rows[b], sems[b])

        def group(g, carry):
            for b in range(NBUF):
                ci = g * NBUF + b
                pltpu.make_async_copy(h_hbm.at[pl.ds(0, K)], rows[b],
                                      sems[b]).wait()
                pltpu.make_async_copy(dst_hbm.at[pl.ds(0, K)], idx[b][1],
                                      isems[b]).wait()
                pltpu.sync_copy(rows[b], acc.at[idx[b][1]], add=True)
                if with_deg:
                    pltpu.sync_copy(ones_v, dacc.at[idx[b][1]], add=True)

                @pl.when(ci + NBUF < CHUNKS)
                def _(b=b, ci=ci):
                    load_idx(b, ci + NBUF)
                    pltpu.async_copy(h_hbm.at[idx[b][0]], rows[b], sems[b])
            return carry

        lax.fori_loop(0, GROUPS, group, 0)

        plsc.subcore_barrier()
        pltpu.sync_copy(acc.at[pl.ds(r0, ROWS_PER_TILE)],
                        out_hbm.at[cid, pl.ds(r0, ROWS_PER_TILE)])

        @pl.when(sid == 0)
        def _():
            pltpu.sync_copy(acc.at[pl.ds(NS * ROWS_PER_TILE, ROWS_TAIL)],
                            out_hbm.at[cid, pl.ds(NS * ROWS_PER_TILE, ROWS_TAIL)])
        if with_deg:
            @pl.when(cid == 0)
            def _():
                pltpu.sync_copy(dacc.at[pl.ds(d0, DEG_CHUNK)],
                                deg0_hbm.at[pl.ds(d0, DEG_CHUNK)])

            @pl.when(cid == 1)
            def _():
                pltpu.sync_copy(dacc.at[pl.ds(d0, DEG_CHUNK)],
                                deg1_hbm.at[pl.ds(d0, DEG_CHUNK)])

    return segsum


_SEGSUM_CACHE = {}


def _segsum_kernel(with_deg: bool):
    if with_deg not in _SEGSUM_CACHE:
        _SEGSUM_CACHE[with_deg] = _make_segsum(with_deg)
    return _SEGSUM_CACHE[with_deg]


BLK = 1000


def _encode_body(x_ref, aa_ref, ab_ref, da_ref, db_ref, wd_ref, wb_ref, b_ref,
                 out_ref):
    deg = da_ref[...] + db_ref[...]
    agg = aa_ref[...] + ab_ref[...]
    h = (deg * (jnp.dot(x_ref[...], wd_ref[...],
                        preferred_element_type=jnp.float32) + b_ref[...])
         + jnp.dot(agg, wb_ref[...], preferred_element_type=jnp.float32))
    out_ref[...] = jnp.where(h > 0, h, 0.2 * h)


def _stage2_body(h_ref, aa_ref, ab_ref, da_ref, db_ref, noise_ref,
                 wmd_ref, wmb_ref, bm_ref, wvd_ref, wvb_ref, bv_ref,
                 wdec_ref, bdec_ref, out_ref):
    deg = da_ref[...] + db_ref[...]
    agg = aa_ref[...] + ab_ref[...]
    h = h_ref[...]
    for r in range(R):
        m = deg * (jnp.dot(h, wmd_ref[r], preferred_element_type=jnp.float32)
                   + bm_ref[r]) + jnp.dot(agg, wmb_ref[r],
                                          preferred_element_type=jnp.float32)
        v = deg * (jnp.dot(h, wvd_ref[r], preferred_element_type=jnp.float32)
                   + bv_ref[r]) + jnp.dot(agg, wvb_ref[r],
                                          preferred_element_type=jnp.float32)
        z = m + noise_ref[:, r, :] * jnp.exp(0.5 * v)
        out_ref[:, r, :] = (jnp.dot(z, wdec_ref[...],
                                    preferred_element_type=jnp.float32)
                            + bdec_ref[...])


def _shuffled_weights(W, b):
    top, bot = W[:C], W[C:]
    wd = (top - bot).reshape(C, C, R).transpose(2, 0, 1)
    wb = bot.reshape(C, C, R).transpose(2, 0, 1)
    bs = b.reshape(C, R).T.reshape(R, 1, C)
    return wd, wb, bs


def kernel(x, edge_index, W_enc, b_enc, W_mean, b_mean, W_logvar, b_logvar,
           W_dec, b_dec):
    pad_src = jnp.arange(K, dtype=edge_index.dtype)
    pad_dst = jnp.arange(N, N + K, dtype=edge_index.dtype)
    src_parts, dst_parts, start = [], [], 0
    for w in range(NW):
        n = 79 if w < 4 else 78
        src_parts += [edge_index[0, start * K:(start + n) * K]]
        dst_parts += [edge_index[1, start * K:(start + n) * K]]
        src_parts += [pad_src] * (CHUNKS - n)
        dst_parts += [pad_dst] * (CHUNKS - n)
        start += n
    src = jnp.concatenate(src_parts).astype(jnp.int32)
    dst = jnp.concatenate(dst_parts).astype(jnp.int32)
    z2 = jnp.zeros((N, C), jnp.float32)
    z1 = jnp.zeros((NPAD,), jnp.float32)

    agg1, deg0, deg1 = _segsum_kernel(True)(x, src, dst, z2, z1)
    deg_a = deg0[:N].reshape(N, 1)
    deg_b = deg1[:N].reshape(N, 1)

    wd_enc = W_enc[:C] - W_enc[C:]
    wb_enc = W_enc[C:]
    full = lambda s: pl.BlockSpec(s, lambda i: (0,) * len(s))
    rows = lambda s: pl.BlockSpec(s, lambda i: (i,) + (0,) * (len(s) - 1))
    h1 = pl.pallas_call(
        _encode_body,
        grid=(N // BLK,),
        in_specs=[rows((BLK, C)), rows((BLK, C)), rows((BLK, C)),
                  rows((BLK, 1)), rows((BLK, 1)),
                  full((C, C)), full((C, C)), full((1, C))],
        out_specs=rows((BLK, C)),
        out_shape=jax.ShapeDtypeStruct((N, C), jnp.float32),
    )(x, agg1[0], agg1[1], deg_a, deg_b, wd_enc, wb_enc, b_enc.reshape(1, C))

    agg2 = _segsum_kernel(False)(h1, src, dst, z2, z1)
    if isinstance(agg2, (list, tuple)):
        agg2 = agg2[0]

    wmd, wmb, bm = _shuffled_weights(W_mean, b_mean)
    wvd, wvb, bv = _shuffled_weights(W_logvar, b_logvar)
    noise = jax.random.normal(jax.random.key(42), (N * R, OUT),
                              jnp.float32).reshape(N, R, OUT)

    out3 = pl.pallas_call(
        _stage2_body,
        grid=(N // BLK,),
        in_specs=[rows((BLK, C)), rows((BLK, C)), rows((BLK, C)),
                  rows((BLK, 1)), rows((BLK, 1)), rows((BLK, R, C)),
                  full((R, C, C)), full((R, C, C)), full((R, 1, C)),
                  full((R, C, C)), full((R, C, C)), full((R, 1, C)),
                  full((C, OUT)), full((1, OUT))],
        out_specs=rows((BLK, R, OUT)),
        out_shape=jax.ShapeDtypeStruct((N, R, OUT), jnp.float32),
    )(h1, agg2[0], agg2[1], deg_a, deg_b, noise,
      wmd, wmb, bm, wvd, wvb, bv, W_dec, b_dec.reshape(1, OUT))

    return out3.reshape(N * R, OUT)


# speedup vs baseline: 2.4013x; 1.0312x over previous
import functools

import jax
import jax.numpy as jnp
from jax import lax
from jax.experimental import pallas as pl
from jax.experimental.pallas import tpu as pltpu
from jax.experimental.pallas import tpu_sc as plsc

N = 10000
E = 320000
C = 128
R = 4
OUT = 128

NC = 2
NS = 16
NW = NC * NS

K = 128
CH_TOT = 2560
EPAD = CH_TOT * K
CHUNKS = CH_TOT // NW
NBUF = 2
GROUPS = CHUNKS // NBUF
ACCN = N + K
ROWS_PER_TILE = 624
ROWS_TAIL = N - NS * ROWS_PER_TILE
NPAD = 10240
DEG_CHUNK = NPAD // NS


def _make_segsum(with_deg: bool):
    mesh = plsc.VectorSubcoreMesh(core_axis_name="c", subcore_axis_name="s")
    out_type = [jax.ShapeDtypeStruct((NC, N, C), jnp.float32)]
    scratch = []
    for _ in range(NBUF):
        scratch += [pltpu.VMEM((K,), jnp.int32),
                    pltpu.VMEM((K,), jnp.int32)]
    scratch += [pltpu.VMEM((K, C), jnp.float32) for _ in range(NBUF)]
    scratch += [pltpu.VMEM_SHARED((ACCN, C), jnp.float32)]
    scratch += [pltpu.SemaphoreType.DMA for _ in range(3 * NBUF)]
    if with_deg:
        out_type.append(jax.ShapeDtypeStruct((NPAD,), jnp.float32))
        out_type.append(jax.ShapeDtypeStruct((NPAD,), jnp.float32))
        scratch += [
            pltpu.VMEM((K,), jnp.float32),
            pltpu.VMEM_SHARED((NPAD,), jnp.float32),
        ]

    @functools.partial(pl.kernel, out_type=out_type, mesh=mesh,
                       scratch_types=scratch)
    def segsum(h_hbm, src_hbm, dst_hbm, z2_hbm, z1_hbm, *rest):
        if with_deg:
            out_hbm, deg0_hbm, deg1_hbm, *rest2 = rest
        else:
            out_hbm, *rest2 = rest
        idx = [(rest2[2 * b], rest2[2 * b + 1]) for b in range(NBUF)]
        rows = rest2[2 * NBUF:3 * NBUF]
        acc = rest2[3 * NBUF]
        sems = rest2[3 * NBUF + 1:4 * NBUF + 1]
        isems = rest2[4 * NBUF + 1:5 * NBUF + 1]
        ssems = rest2[5 * NBUF + 1:6 * NBUF + 1]
        if with_deg:
            ones_v, dacc = rest2[6 * NBUF + 1:]
        cid = lax.axis_index("c")
        sid = lax.axis_index("s")
        wid = sid * NC + cid

        r0 = sid * ROWS_PER_TILE
        pltpu.sync_copy(z2_hbm.at[pl.ds(r0, ROWS_PER_TILE)],
                        acc.at[pl.ds(r0, ROWS_PER_TILE)])

        @pl.when(sid == 0)
        def _():
            pltpu.sync_copy(z2_hbm.at[pl.ds(NS * ROWS_PER_TILE, ROWS_TAIL)],
                            acc.at[pl.ds(NS * ROWS_PER_TILE, ROWS_TAIL)])
        if with_deg:
            d0 = sid * DEG_CHUNK
            pltpu.sync_copy(z1_hbm.at[pl.ds(d0, DEG_CHUNK)],
                            dacc.at[pl.ds(d0, DEG_CHUNK)])
            for i in range(K // 16):
                ones_v[pl.ds(i * 16, 16)] = jnp.ones((16,), jnp.float32)
        plsc.subcore_barrier()

        base = wid * CHUNKS * K

        def load_idx(b, ci):
            pltpu.sync_copy(src_hbm.at[pl.ds(base + ci * K, K)], idx[b][0])
            pltpu.async_copy(dst_hbm.at[pl.ds(base + ci * K, K)], idx[b][1],
                             isems[b])

        for b in range(NBUF):
            load_idx(b, b)
            pltpu.async_copy(h_hbm.at[idx[b][0]], rows[b], sems[b])

        def group(g, carry):
            for b in range(NBUF):
                ci = g * NBUF + b
                pltpu.make_async_copy(h_hbm.at[pl.ds(0, K)], rows[b],
                                      sems[b]).wait()
                pltpu.make_async_copy(dst_hbm.at[pl.ds(0, K)], idx[b][1],
                                      isems[b]).wait()

                @pl.when(ci + NBUF < CHUNKS)
                def _(b=b, ci=ci):
                    pltpu.async_copy(
                        src_hbm.at[pl.ds(base + (ci + NBUF) * K, K)],
                        idx[b][0], ssems[b])

                pltpu.sync_copy(rows[b], acc.at[idx[b][1]], add=True)
                if with_deg:
                    pltpu.sync_copy(ones_v, dacc.at[idx[b][1]], add=True)

                @pl.when(ci + NBUF < CHUNKS)
                def _(b=b, ci=ci):
                    pltpu.make_async_copy(src_hbm.at[pl.ds(0, K)], idx[b][0],
                                          ssems[b]).wait()
                    pltpu.async_copy(h_hbm.at[idx[b][0]], rows[b], sems[b])
                    pltpu.async_copy(dst_hbm.at[pl.ds(base + (ci + NBUF) * K,
                                                      K)],
                                     idx[b][1], isems[b])
            return carry

        lax.fori_loop(0, GROUPS, group, 0)

        plsc.subcore_barrier()
        pltpu.sync_copy(acc.at[pl.ds(r0, ROWS_PER_TILE)],
                        out_hbm.at[cid, pl.ds(r0, ROWS_PER_TILE)])

        @pl.when(sid == 0)
        def _():
            pltpu.sync_copy(acc.at[pl.ds(NS * ROWS_PER_TILE, ROWS_TAIL)],
                            out_hbm.at[cid, pl.ds(NS * ROWS_PER_TILE, ROWS_TAIL)])
        if with_deg:
            @pl.when(cid == 0)
            def _():
                pltpu.sync_copy(dacc.at[pl.ds(d0, DEG_CHUNK)],
                                deg0_hbm.at[pl.ds(d0, DEG_CHUNK)])

            @pl.when(cid == 1)
            def _():
                pltpu.sync_copy(dacc.at[pl.ds(d0, DEG_CHUNK)],
                                deg1_hbm.at[pl.ds(d0, DEG_CHUNK)])

    return segsum


_SEGSUM_CACHE = {}


def _segsum_kernel(with_deg: bool):
    if with_deg not in _SEGSUM_CACHE:
        _SEGSUM_CACHE[with_deg] = _make_segsum(with_deg)
    return _SEGSUM_CACHE[with_deg]


BLK = 1000


def _encode_body(x_ref, aa_ref, ab_ref, da_ref, db_ref, wd_ref, wb_ref, b_ref,
                 out_ref):
    deg = da_ref[...] + db_ref[...]
    agg = aa_ref[...] + ab_ref[...]
    h = (deg * (jnp.dot(x_ref[...], wd_ref[...],
                        preferred_element_type=jnp.float32) + b_ref[...])
         + jnp.dot(agg, wb_ref[...], preferred_element_type=jnp.float32))
    out_ref[...] = jnp.where(h > 0, h, 0.2 * h)


def _stage2_body(h_ref, aa_ref, ab_ref, da_ref, db_ref, noise_ref,
                 wmd_ref, wmb_ref, bm_ref, wvd_ref, wvb_ref, bv_ref,
                 wdec_ref, bdec_ref, out_ref):
    deg = da_ref[...] + db_ref[...]
    agg = aa_ref[...] + ab_ref[...]
    h = h_ref[...]
    for r in range(R):
        m = deg * (jnp.dot(h, wmd_ref[r], preferred_element_type=jnp.float32)
                   + bm_ref[r]) + jnp.dot(agg, wmb_ref[r],
                                          preferred_element_type=jnp.float32)
        v = deg * (jnp.dot(h, wvd_ref[r], preferred_element_type=jnp.float32)
                   + bv_ref[r]) + jnp.dot(agg, wvb_ref[r],
                                          preferred_element_type=jnp.float32)
        z = m + noise_ref[:, r, :] * jnp.exp(0.5 * v)
        out_ref[:, r, :] = (jnp.dot(z, wdec_ref[...],
                                    preferred_element_type=jnp.float32)
                            + bdec_ref[...])


def _shuffled_weights(W, b):
    top, bot = W[:C], W[C:]
    wd = (top - bot).reshape(C, C, R).transpose(2, 0, 1)
    wb = bot.reshape(C, C, R).transpose(2, 0, 1)
    bs = b.reshape(C, R).T.reshape(R, 1, C)
    return wd, wb, bs


def kernel(x, edge_index, W_enc, b_enc, W_mean, b_mean, W_logvar, b_logvar,
           W_dec, b_dec):
    pad_src = jnp.arange(K, dtype=edge_index.dtype)
    pad_dst = jnp.arange(N, N + K, dtype=edge_index.dtype)
    src_parts, dst_parts, start = [], [], 0
    for w in range(NW):
        n = 79 if w < 4 else 78
        src_parts += [edge_index[0, start * K:(start + n) * K]]
        dst_parts += [edge_index[1, start * K:(start + n) * K]]
        src_parts += [pad_src] * (CHUNKS - n)
        dst_parts += [pad_dst] * (CHUNKS - n)
        start += n
    src = jnp.concatenate(src_parts).astype(jnp.int32)
    dst = jnp.concatenate(dst_parts).astype(jnp.int32)
    z2 = jnp.zeros((N, C), jnp.float32)
    z1 = jnp.zeros((NPAD,), jnp.float32)

    agg1, deg0, deg1 = _segsum_kernel(True)(x, src, dst, z2, z1)
    deg_a = deg0[:N].reshape(N, 1)
    deg_b = deg1[:N].reshape(N, 1)

    wd_enc = W_enc[:C] - W_enc[C:]
    wb_enc = W_enc[C:]
    full = lambda s: pl.BlockSpec(s, lambda i: (0,) * len(s))
    rows = lambda s: pl.BlockSpec(s, lambda i: (i,) + (0,) * (len(s) - 1))
    h1 = pl.pallas_call(
        _encode_body,
        grid=(N // BLK,),
        in_specs=[rows((BLK, C)), rows((BLK, C)), rows((BLK, C)),
                  rows((BLK, 1)), rows((BLK, 1)),
                  full((C, C)), full((C, C)), full((1, C))],
        out_specs=rows((BLK, C)),
        out_shape=jax.ShapeDtypeStruct((N, C), jnp.float32),
    )(x, agg1[0], agg1[1], deg_a, deg_b, wd_enc, wb_enc, b_enc.reshape(1, C))

    agg2 = _segsum_kernel(False)(h1, src, dst, z2, z1)
    if isinstance(agg2, (list, tuple)):
        agg2 = agg2[0]

    wmd, wmb, bm = _shuffled_weights(W_mean, b_mean)
    wvd, wvb, bv = _shuffled_weights(W_logvar, b_logvar)
    noise = jax.random.normal(jax.random.key(42), (N * R, OUT),
                              jnp.float32).reshape(N, R, OUT)

    out3 = pl.pallas_call(
        _stage2_body,
        grid=(N // BLK,),
        in_specs=[rows((BLK, C)), rows((BLK, C)), rows((BLK, C)),
                  rows((BLK, 1)), rows((BLK, 1)), rows((BLK, R, C)),
                  full((R, C, C)), full((R, C, C)), full((R, 1, C)),
                  full((R, C, C)), full((R, C, C)), full((R, 1, C)),
                  full((C, OUT)), full((1, OUT))],
        out_specs=rows((BLK, R, OUT)),
        out_shape=jax.ShapeDtypeStruct((N, R, OUT), jnp.float32),
    )(h1, agg2[0], agg2[1], deg_a, deg_b, noise,
      wmd, wmb, bm, wvd, wvb, bv, W_dec, b_dec.reshape(1, OUT))

    return out3.reshape(N * R, OUT)
